# baseline (speedup 1.0000x reference)
import functools

import jax
import jax.numpy as jnp
from jax import lax
from jax.experimental import pallas as pl
from jax.experimental.pallas import tpu as pltpu
from jax.experimental.pallas import tpu_sc as plsc

N = 10000
E = 320000
EMB = 128
HEADS = 4
HD = HEADS * EMB

NW = 32
EW = E // NW
ET = E // 16
NSL = N // 16
K = 32
NCH = ET // K
NBUF = 5

f32 = jnp.float32
i32 = jnp.int32


def _prelu(z, a):
    return jnp.where(z >= 0, z, a * z)


def _scal(v):
    return jnp.reshape(v.astype(f32), (1, 1))


def _k_embed(x_ref, w_ref, a_ref, o_ref):
    o_ref[...] = _prelu(
        jnp.dot(x_ref[...], w_ref[...], preferred_element_type=f32),
        a_ref[0, 0])


def _tc_embed(x, wt, a):
    return pl.pallas_call(
        _k_embed,
        grid=(10,),
        in_specs=[
            pl.BlockSpec((N // 10, 128), lambda i: (i, 0)),
            pl.BlockSpec((128, 128), lambda i: (0, 0)),
            pl.BlockSpec((1, 1), lambda i: (0, 0)),
        ],
        out_specs=pl.BlockSpec((N // 10, 128), lambda i: (i, 0)),
        out_shape=jax.ShapeDtypeStruct((N, 128), f32),
    )(x, wt, a)


def _pack_bf16_pairs(v):
    v32 = v.reshape(v.shape[:-1] + (EMB // 32, 32))
    lo = v32[..., :16].astype(jnp.bfloat16)
    hi = v32[..., 16:].astype(jnp.bfloat16)
    lo32 = lax.bitcast_convert_type(lo, jnp.uint16).astype(jnp.uint32)
    hi32 = lax.bitcast_convert_type(hi, jnp.uint16).astype(jnp.uint32)
    w = lo32 | (hi32 << 16)
    return lax.bitcast_convert_type(
        w.reshape(v.shape[:-1] + (EMB // 2,)), i32)


def _unpack_view(x_i32):
    b = lax.bitcast_convert_type(x_i32, jnp.bfloat16)
    return b.reshape(x_i32.shape[0] * x_i32.shape[1], EMB)


def _k_edge(ea_ref, w_ref, b_ref, attj_ref, a_ref, em_ref, t_ref):
    emt = _prelu(
        jnp.dot(ea_ref[...], w_ref[...], preferred_element_type=f32)
        + b_ref[...], a_ref[0, 0])
    em4 = emt.reshape(emt.shape[0], HEADS, EMB)
    t_ref[...] = jnp.sum(em4 * attj_ref[...][None], axis=-1).T
    em_ref[...] = jnp.transpose(_pack_bf16_pairs(em4), (1, 0, 2))


def _tc_edge(ea, wt, b, attj, a, te=1280):
    return pl.pallas_call(
        _k_edge,
        grid=(E // te,),
        in_specs=[
            pl.BlockSpec((te, 16), lambda i: (i, 0)),
            pl.BlockSpec((16, HD), lambda i: (0, 0)),
            pl.BlockSpec((1, HD), lambda i: (0, 0)),
            pl.BlockSpec((HEADS, EMB), lambda i: (0, 0)),
            pl.BlockSpec((1, 1), lambda i: (0, 0)),
        ],
        out_specs=[
            pl.BlockSpec((HEADS, te, EMB // 2), lambda i: (0, i, 0)),
            pl.BlockSpec((HEADS, te), lambda i: (0, i)),
        ],
        out_shape=[
            jax.ShapeDtypeStruct((HEADS, E, EMB // 2), i32),
            jax.ShapeDtypeStruct((HEADS, E), f32),
        ],
    )(ea, wt, b, attj, a)


def _k_xh(h_ref, w_ref, b_ref, a_ref, xh_ref, xhp_ref):
    xht = _prelu(
        jnp.dot(h_ref[...], w_ref[...], preferred_element_type=f32)
        + b_ref[...], a_ref[0, 0])
    xh4 = xht.reshape(xht.shape[0], HEADS, EMB)
    xh_ref[...] = jnp.transpose(xh4, (1, 0, 2))
    xhp_ref[...] = jnp.transpose(_pack_bf16_pairs(xh4), (1, 0, 2))


def _tc_xh(h, wt, b, a, tn=1000):
    return pl.pallas_call(
        _k_xh,
        grid=(N // tn,),
        in_specs=[
            pl.BlockSpec((tn, 128), lambda i: (i, 0)),
            pl.BlockSpec((128, HD), lambda i: (0, 0)),
            pl.BlockSpec((1, HD), lambda i: (0, 0)),
            pl.BlockSpec((1, 1), lambda i: (0, 0)),
        ],
        out_specs=[
            pl.BlockSpec((HEADS, tn, EMB), lambda i: (0, i, 0)),
            pl.BlockSpec((HEADS, tn, EMB // 2), lambda i: (0, i, 0)),
        ],
        out_shape=[
            jax.ShapeDtypeStruct((HEADS, N, EMB), f32),
            jax.ShapeDtypeStruct((HEADS, N, EMB // 2), i32),
        ],
    )(h, wt, b, a)


def _k_sij(xh_ref, atti_ref, attj_ref, si_ref, sj_ref):
    xh = xh_ref[...]
    si_ref[...] = jnp.sum(xh * atti_ref[...][:, None, :], axis=-1)
    sj_ref[...] = jnp.sum(xh * attj_ref[...][:, None, :], axis=-1)


def _tc_sij(xh, atti, attj):
    return pl.pallas_call(
        _k_sij,
        grid=(1,),
        compiler_params=pltpu.CompilerParams(
            vmem_limit_bytes=100 * 1024 * 1024),
        in_specs=[
            pl.BlockSpec((HEADS, N, EMB), lambda i: (0, 0, 0)),
            pl.BlockSpec((HEADS, EMB), lambda i: (0, 0)),
            pl.BlockSpec((HEADS, EMB), lambda i: (0, 0)),
        ],
        out_specs=[
            pl.BlockSpec((HEADS, N), lambda i: (0, 0)),
            pl.BlockSpec((HEADS, N), lambda i: (0, 0)),
        ],
        out_shape=[
            jax.ShapeDtypeStruct((HEADS, N), f32),
            jax.ShapeDtypeStruct((HEADS, N), f32),
        ],
    )(xh, atti, attj)


def _k_prep(si_ref, sj_ref, t_ref, a_ref, m_ref, exs_ref):
    a = a_ref[0, 0]
    si = si_ref[...]
    sj = sj_ref[...]
    msi = jnp.max(si, axis=1, keepdims=True)
    mt = jnp.maximum(jnp.max(t_ref[...], axis=1, keepdims=True), 0.0)
    m = _prelu(sj + msi + mt, a)
    m_ref[...] = m
    exs_ref[...] = jnp.exp(_prelu(si + sj, a) - m)


def _tc_prep(si, sj, t, a):
    return pl.pallas_call(
        _k_prep,
        grid=(1,),
        in_specs=[
            pl.BlockSpec((HEADS, N), lambda i: (0, 0)),
            pl.BlockSpec((HEADS, N), lambda i: (0, 0)),
            pl.BlockSpec((HEADS, E), lambda i: (0, 0)),
            pl.BlockSpec((1, 1), lambda i: (0, 0)),
        ],
        out_specs=[
            pl.BlockSpec((HEADS, N), lambda i: (0, 0)),
            pl.BlockSpec((HEADS, N), lambda i: (0, 0)),
        ],
        out_shape=[
            jax.ShapeDtypeStruct((HEADS, N), f32),
            jax.ShapeDtypeStruct((HEADS, N), f32),
        ],
    )(si, sj, t, a)


def _k_finish(parts_ref, exs_ref, xh_ref, inv_ref, init_ref):
    den = jnp.sum(parts_ref[...], axis=0) + exs_ref[...]
    inv = 1.0 / (den + 1e-16)
    inv_ref[...] = inv
    init_ref[...] = (exs_ref[...] * inv)[:, :, None] * xh_ref[...]


def _tc_finish(parts, exs, xh):
    return pl.pallas_call(
        _k_finish,
        grid=(1,),
        in_specs=[
            pl.BlockSpec((NW, HEADS, N), lambda i: (0, 0, 0)),
            pl.BlockSpec((HEADS, N), lambda i: (0, 0)),
            pl.BlockSpec((HEADS, N, EMB), lambda i: (0, 0, 0)),
        ],
        out_specs=[
            pl.BlockSpec((HEADS, N), lambda i: (0, 0)),
            pl.BlockSpec((HEADS, N, EMB), lambda i: (0, 0, 0)),
        ],
        out_shape=[
            jax.ShapeDtypeStruct((HEADS, N), f32),
            jax.ShapeDtypeStruct((HEADS, N, EMB), f32),
        ],
    )(parts, exs, xh)


def _k_out(aggr_ref, b_ref, lw_ref, lb_ref, ag_ref, o_ref, *, last):
    out = jnp.mean(aggr_ref[...], axis=0) + b_ref[...]
    mu = jnp.mean(out, axis=-1, keepdims=True)
    var = jnp.mean((out - mu) ** 2, axis=-1, keepdims=True)
    out = (out - mu) / jnp.sqrt(var + 1e-5) * lw_ref[...] + lb_ref[...]
    if not last:
        out = _prelu(out, ag_ref[0, 0])
    o_ref[...] = out


def _tc_out(aggr, b, lw, lb, ag, last, tn=1000):
    return pl.pallas_call(
        functools.partial(_k_out, last=last),
        grid=(N // tn,),
        in_specs=[
            pl.BlockSpec((HEADS, tn, EMB), lambda i: (0, i, 0)),
            pl.BlockSpec((1, EMB), lambda i: (0, 0)),
            pl.BlockSpec((1, EMB), lambda i: (0, 0)),
            pl.BlockSpec((1, EMB), lambda i: (0, 0)),
            pl.BlockSpec((1, 1), lambda i: (0, 0)),
        ],
        out_specs=pl.BlockSpec((tn, EMB), lambda i: (i, 0)),
        out_shape=jax.ShapeDtypeStruct((N, EMB), f32),
    )(aggr, b, lw, lb, ag)


def _sc_denom(src, dst, t, si, sj, m, a16):
    mesh = plsc.VectorSubcoreMesh(core_axis_name="c", subcore_axis_name="s")

    def body(src_h, dst_h, t_h, si_h, sj_h, m_h, a_h,
             ex_h, parts_h,
             src_v, dst_v, t_v, si_v, sj_v, m_v, den_v, ex_v, a_v):
        c = lax.axis_index("c")
        s = lax.axis_index("s")
        w = c * 16 + s
        e0 = w * EW
        pltpu.sync_copy(src_h.at[pl.ds(e0, EW)], src_v)
        pltpu.sync_copy(dst_h.at[pl.ds(e0, EW)], dst_v)
        pltpu.sync_copy(a_h, a_v)
        av = a_v[...]

        for hh in range(HEADS):
            pltpu.sync_copy(si_h.at[pl.ds(hh * N, N)], si_v)
            pltpu.sync_copy(sj_h.at[pl.ds(hh * N, N)], sj_v)
            pltpu.sync_copy(m_h.at[pl.ds(hh * N, N)], m_v)
            pltpu.sync_copy(t_h.at[pl.ds(hh * E + e0, EW)], t_v)

            def zero_body(i, _):
                den_v[pl.ds(i * 16, 16)] = jnp.zeros((16,), f32)
                return 0
            lax.fori_loop(0, N // 16, zero_body, 0)

            def edge_body(g, _):
                b = g * 16
                s16 = src_v[pl.ds(b, 16)]
                d16 = dst_v[pl.ds(b, 16)]
                tt = t_v[pl.ds(b, 16)]
                sid = plsc.load_gather(si_v, [d16])
                sjs = plsc.load_gather(sj_v, [s16])
                ms = plsc.load_gather(m_v, [s16])
                logit = sid + sjs + tt
                alpha = jnp.where(logit >= 0, logit, av * logit)
                e16 = jnp.exp(alpha - ms)
                ex_v[pl.ds(b, 16)] = e16
                plsc.addupdate_scatter(den_v, [s16], e16)
                return 0
            lax.fori_loop(0, EW // 16, edge_body, 0)

            pltpu.sync_copy(ex_v, ex_h.at[pl.ds(hh * E + e0, EW)])
            pltpu.sync_copy(den_v, parts_h.at[pl.ds((w * HEADS + hh) * N, N)])

    kern = pl.kernel(
        body,
        out_type=[
            jax.ShapeDtypeStruct((HEADS * E,), f32),
            jax.ShapeDtypeStruct((NW * HEADS * N,), f32),
        ],
        mesh=mesh,
        compiler_params=pltpu.CompilerParams(needs_layout_passes=False, use_tc_tiling_on_sc=False),
        scratch_types=[
            pltpu.VMEM((EW,), i32),
            pltpu.VMEM((EW,), i32),
            pltpu.VMEM((EW,), f32),
            pltpu.VMEM((N,), f32),
            pltpu.VMEM((N,), f32),
            pltpu.VMEM((N,), f32),
            pltpu.VMEM((N,), f32),
            pltpu.VMEM((EW,), f32),
            pltpu.VMEM((16,), f32),
        ],
    )
    return kern(src, dst, t, si, sj, m, a16)


def _sc_aggr(src, dst, ex, inv, xh2, em2, init2):
    mesh = plsc.VectorSubcoreMesh(core_axis_name="c", subcore_axis_name="s")

    def body(src_h, dst_h, ex_h, inv_h, xh_h, em_h, init_h,
             out_h,
             inv_v, wbuf,
             srcb0, srcb1, srcb2, srcb3, srcb4,
             dstb0, dstb1, dstb2, dstb3, dstb4,
             idxb0, idxb1, idxb2, idxb3, idxb4,
             exb0, exb1, exb2, exb3, exb4,
             emb0, emb1, emb2, emb3, emb4,
             outb0, outb1, outb2, outb3, outb4, spm,
             sin0, sin1, sin2, sin3, sin4,
             sbig0, sbig1, sbig2, sbig3, sbig4,
             ssc0, ssc1, ssc2, ssc3, ssc4):
        c = lax.axis_index("c")
        s = lax.axis_index("s")
        srcb = [srcb0, srcb1, srcb2, srcb3, srcb4]
        dstb = [dstb0, dstb1, dstb2, dstb3, dstb4]
        idxb = [idxb0, idxb1, idxb2, idxb3, idxb4]
        exb = [exb0, exb1, exb2, exb3, exb4]
        emb = [emb0, emb1, emb2, emb3, emb4]
        outb = [outb0, outb1, outb2, outb3, outb4]
        sin = [sin0, sin1, sin2, sin3, sin4]
        sbig = [sbig0, sbig1, sbig2, sbig3, sbig4]
        ssc = [ssc0, ssc1, ssc2, ssc3, ssc4]

        ebase = s * ET

        for ph in range(2):
            hsel = c * 2 + ph
            hN = hsel * N
            hE = hsel * E
            pltpu.sync_copy(inv_h.at[pl.ds(hN, N)], inv_v)
            pltpu.sync_copy(init_h.at[pl.ds(hN + s * NSL, NSL)],
                            spm.at[pl.ds(s * NSL, NSL)])
            plsc.subcore_barrier()

            def e_of(j):
                return ebase + j * K

            def issue_in(j, sl):
                pltpu.async_copy(src_h.at[pl.ds(e_of(j), K)], srcb[sl],
                                 sin[sl])
                pltpu.async_copy(dst_h.at[pl.ds(e_of(j), K)], dstb[sl],
                                 sin[sl])
                pltpu.async_copy(ex_h.at[pl.ds(hE + e_of(j), K)], exb[sl],
                                 sin[sl])
                pltpu.async_copy(em_h.at[pl.ds(hE + e_of(j), K)], emb[sl],
                                 sin[sl])

            def wait_in(j, sl):
                pltpu.make_async_copy(src_h.at[pl.ds(e_of(j), K)], srcb[sl],
                                      sin[sl]).wait()
                pltpu.make_async_copy(dst_h.at[pl.ds(e_of(j), K)], dstb[sl],
                                      sin[sl]).wait()
                pltpu.make_async_copy(ex_h.at[pl.ds(hE + e_of(j), K)],
                                      exb[sl], sin[sl]).wait()
                pltpu.make_async_copy(em_h.at[pl.ds(hE + e_of(j), K)],
                                      emb[sl], sin[sl]).wait()

            def build_issue_big(j, sl):
                hNv = jnp.broadcast_to(hN.astype(i32), (16,))
                for q in range(K // 16):
                    idxb[sl][pl.ds(q * 16, 16)] = (
                        srcb[sl][pl.ds(q * 16, 16)] + hNv)
                pltpu.async_copy(xh_h.at[idxb[sl]], emb[sl], sbig[sl],
                                 add=True)

            def wait_big(j, sl):
                pltpu.make_async_copy(xh_h.at[idxb[sl]], emb[sl],
                                      sbig[sl]).wait()

            def wait_sc(sl):
                pltpu.make_async_copy(outb[sl], spm.at[dstb[sl]],
                                      ssc[sl]).wait()

            def compute_scatter(j, sl):
                for q in range(K // 16):
                    b = q * 16
                    s16 = srcb[sl][pl.ds(b, 16)]
                    ex16 = exb[sl][pl.ds(b, 16)]
                    iv16 = plsc.load_gather(inv_v, [s16])
                    wbuf[pl.ds(b, 16)] = ex16 * iv16

                def row_body(r, _):
                    for u in range(2):
                        jj = r * 2 + u
                        wv = plsc.load_gather(
                            wbuf, [jnp.broadcast_to(jj, (16,))])
                        for g in range(EMB // 32):
                            pair = emb[sl][jj, pl.ds(g * 32, 32)]
                            av, bv = plsc.unpack(
                                pair, format=plsc.PackFormat.INTERLEAVED)
                            outb[sl][jj, pl.ds(g * 32, 16)] = av * wv
                            outb[sl][jj, pl.ds(g * 32 + 16, 16)] = bv * wv
                    return 0
                lax.fori_loop(0, K // 2, row_body, 0)
                pltpu.async_copy(outb[sl], spm.at[dstb[sl]], ssc[sl],
                                 add=True)

            issue_in(0, 0)
            issue_in(1, 1)
            wait_in(0, 0)
            build_issue_big(0, 0)

            def outer(o, _):
                for b5 in range(NBUF):
                    j = o * NBUF + b5
                    sl = b5
                    sl2 = (b5 + 2) % NBUF
                    sl1 = (b5 + 1) % NBUF

                    @pl.when(jnp.logical_and(j >= 3, j + 2 < NCH))
                    def _():
                        wait_sc(sl2)

                    @pl.when(j + 2 < NCH)
                    def _():
                        issue_in(j + 2, sl2)

                    @pl.when(j + 1 < NCH)
                    def _():
                        wait_in(j + 1, sl1)
                        build_issue_big(j + 1, sl1)

                    @pl.when(j < NCH)
                    def _():
                        wait_big(j, sl)
                        compute_scatter(j, sl)
                return 0
            lax.fori_loop(0, (NCH + NBUF - 1) // NBUF, outer, 0)

            for sl in range(NBUF):
                wait_sc(sl)

            plsc.subcore_barrier()
            pltpu.sync_copy(spm.at[pl.ds(s * NSL, NSL)],
                            out_h.at[pl.ds(hN + s * NSL, NSL)])
            plsc.subcore_barrier()

    kern = pl.kernel(
        body,
        out_type=[jax.ShapeDtypeStruct((HEADS * N, EMB), f32)],
        mesh=mesh,
        compiler_params=pltpu.CompilerParams(needs_layout_passes=False,
                                             use_tc_tiling_on_sc=False),
        scratch_types=(
            [pltpu.VMEM((N,), f32), pltpu.VMEM((K,), f32)]
            + [pltpu.VMEM((K,), i32) for _ in range(NBUF)]
            + [pltpu.VMEM((K,), i32) for _ in range(NBUF)]
            + [pltpu.VMEM((K,), i32) for _ in range(NBUF)]
            + [pltpu.VMEM((K,), f32) for _ in range(NBUF)]
            + [pltpu.VMEM((K, EMB), jnp.bfloat16)
               for _ in range(NBUF)]
            + [pltpu.VMEM((K, EMB), f32) for _ in range(NBUF)]
            + [pltpu.VMEM_SHARED((N, EMB), f32)]
            + [pltpu.SemaphoreType.DMA for _ in range(3 * NBUF)]
        ),
    )
    (out,) = kern(src, dst, ex, inv, xh2, em2, init2)
    return out


def kernel(x, edge_index, edge_attr, params):
    src = edge_index[0]
    dst = edge_index[1]
    ea = edge_attr.astype(f32)
    ag = params['a_gnn']

    h = _tc_embed(x, params['x_emb_W'].T, _scal(ag))

    nlayers = len(params['layers'])
    for li, p in enumerate(params['layers']):
        a = p['a']
        att = p['att'][0]
        atti = att[:, :EMB]
        attj = att[:, EMB:]
        em3, t = _tc_edge(ea, p['We_w'].T, p['We_b'][None, :], attj,
                          _scal(a))
        xh3, xhp = _tc_xh(h, p['Wlin_w'].T, p['Wlin_b'][None, :], _scal(a))
        si, sj = _tc_sij(xh3, atti, attj)
        m, exs = _tc_prep(si, sj, t, _scal(a))
        ex, parts = _sc_denom(src, dst, t.reshape(-1), si.reshape(-1),
                              sj.reshape(-1), m.reshape(-1),
                              jnp.full((16,), a, f32))
        inv, init = _tc_finish(parts.reshape(NW, HEADS, N), exs, xh3)
        aggr = _sc_aggr(src, dst, ex, inv.reshape(-1),
                        _unpack_view(xhp),
                        _unpack_view(em3),
                        init.reshape(HEADS * N, EMB))
        h = _tc_out(aggr.reshape(HEADS, N, EMB), p['bias'][None, :],
                    p['ln_w'][None, :], p['ln_b'][None, :], _scal(ag),
                    last=(li == nlayers - 1))
    return h


# speedup vs baseline: 2.9908x; 2.9908x over previous
import functools

import jax
import jax.numpy as jnp
from jax import lax
from jax.experimental import pallas as pl
from jax.experimental.pallas import tpu as pltpu
from jax.experimental.pallas import tpu_sc as plsc

N = 10000
E = 320000
EMB = 128
HEADS = 4
HD = HEADS * EMB

NW = 32
EW = E // NW
ET = E // 16
NSL = N // 16
K = 32
NCH = ET // K
NBUF = 5

f32 = jnp.float32
i32 = jnp.int32


def _prelu(z, a):
    return jnp.where(z >= 0, z, a * z)


def _scal(v):
    return jnp.reshape(v.astype(f32), (1, 1))


def _k_embed(x_ref, w_ref, a_ref, o_ref):
    o_ref[...] = _prelu(
        jnp.dot(x_ref[...], w_ref[...], preferred_element_type=f32),
        a_ref[0, 0])


def _tc_embed(x, wt, a):
    return pl.pallas_call(
        _k_embed,
        grid=(10,),
        in_specs=[
            pl.BlockSpec((N // 10, 128), lambda i: (i, 0)),
            pl.BlockSpec((128, 128), lambda i: (0, 0)),
            pl.BlockSpec((1, 1), lambda i: (0, 0)),
        ],
        out_specs=pl.BlockSpec((N // 10, 128), lambda i: (i, 0)),
        out_shape=jax.ShapeDtypeStruct((N, 128), f32),
    )(x, wt, a)


import numpy as _np

_PERM = _np.zeros((EMB, EMB), dtype=_np.float32)
for _g in range(EMB // 32):
    for _t in range(16):
        _PERM[_g * 32 + _t, _g * 32 + 2 * _t] = 1.0
        _PERM[_g * 32 + 16 + _t, _g * 32 + 2 * _t + 1] = 1.0


def _pack_bf16_pairs(v, pm):
    return jnp.dot(v, pm, preferred_element_type=f32).astype(jnp.bfloat16)


def _k_edge(ea_ref, w_ref, b_ref, attj_ref, a_ref, pm_ref, em_ref, t_ref):
    emt = _prelu(
        jnp.dot(ea_ref[...], w_ref[...], preferred_element_type=f32)
        + b_ref[...], a_ref[0, 0])
    em4 = emt.reshape(emt.shape[0], HEADS, EMB)
    t_ref[...] = jnp.sum(em4 * attj_ref[...][None], axis=-1).T
    em_ref[...] = jnp.transpose(_pack_bf16_pairs(em4, pm_ref[...]), (1, 0, 2))


def _tc_edge(ea, wt, b, attj, a, te=1280):
    return pl.pallas_call(
        _k_edge,
        grid=(E // te,),
        in_specs=[
            pl.BlockSpec((te, 16), lambda i: (i, 0)),
            pl.BlockSpec((16, HD), lambda i: (0, 0)),
            pl.BlockSpec((1, HD), lambda i: (0, 0)),
            pl.BlockSpec((HEADS, EMB), lambda i: (0, 0)),
            pl.BlockSpec((1, 1), lambda i: (0, 0)),
            pl.BlockSpec((EMB, EMB), lambda i: (0, 0)),
        ],
        out_specs=[
            pl.BlockSpec((HEADS, te, EMB), lambda i: (0, i, 0)),
            pl.BlockSpec((HEADS, te), lambda i: (0, i)),
        ],
        out_shape=[
            jax.ShapeDtypeStruct((HEADS, E, EMB), jnp.bfloat16),
            jax.ShapeDtypeStruct((HEADS, E), f32),
        ],
    )(ea, wt, b, attj, a, jnp.asarray(_PERM))


def _k_xh(h_ref, w_ref, b_ref, a_ref, pm_ref, xh_ref, xhp_ref):
    xht = _prelu(
        jnp.dot(h_ref[...], w_ref[...], preferred_element_type=f32)
        + b_ref[...], a_ref[0, 0])
    xh4 = xht.reshape(xht.shape[0], HEADS, EMB)
    xh_ref[...] = jnp.transpose(xh4, (1, 0, 2))
    xhp_ref[...] = jnp.transpose(_pack_bf16_pairs(xh4, pm_ref[...]),
                                 (1, 0, 2))


def _tc_xh(h, wt, b, a, tn=1000):
    return pl.pallas_call(
        _k_xh,
        grid=(N // tn,),
        in_specs=[
            pl.BlockSpec((tn, 128), lambda i: (i, 0)),
            pl.BlockSpec((128, HD), lambda i: (0, 0)),
            pl.BlockSpec((1, HD), lambda i: (0, 0)),
            pl.BlockSpec((1, 1), lambda i: (0, 0)),
            pl.BlockSpec((EMB, EMB), lambda i: (0, 0)),
        ],
        out_specs=[
            pl.BlockSpec((HEADS, tn, EMB), lambda i: (0, i, 0)),
            pl.BlockSpec((HEADS, tn, EMB), lambda i: (0, i, 0)),
        ],
        out_shape=[
            jax.ShapeDtypeStruct((HEADS, N, EMB), f32),
            jax.ShapeDtypeStruct((HEADS, N, EMB), jnp.bfloat16),
        ],
    )(h, wt, b, a, jnp.asarray(_PERM))


def _k_sij(xh_ref, atti_ref, attj_ref, si_ref, sj_ref):
    xh = xh_ref[...]
    si_ref[...] = jnp.sum(xh * atti_ref[...][:, None, :], axis=-1)
    sj_ref[...] = jnp.sum(xh * attj_ref[...][:, None, :], axis=-1)


def _tc_sij(xh, atti, attj):
    return pl.pallas_call(
        _k_sij,
        grid=(1,),
        compiler_params=pltpu.CompilerParams(
            vmem_limit_bytes=100 * 1024 * 1024),
        in_specs=[
            pl.BlockSpec((HEADS, N, EMB), lambda i: (0, 0, 0)),
            pl.BlockSpec((HEADS, EMB), lambda i: (0, 0)),
            pl.BlockSpec((HEADS, EMB), lambda i: (0, 0)),
        ],
        out_specs=[
            pl.BlockSpec((HEADS, N), lambda i: (0, 0)),
            pl.BlockSpec((HEADS, N), lambda i: (0, 0)),
        ],
        out_shape=[
            jax.ShapeDtypeStruct((HEADS, N), f32),
            jax.ShapeDtypeStruct((HEADS, N), f32),
        ],
    )(xh, atti, attj)


def _k_prep(si_ref, sj_ref, t_ref, a_ref, m_ref, exs_ref):
    a = a_ref[0, 0]
    si = si_ref[...]
    sj = sj_ref[...]
    msi = jnp.max(si, axis=1, keepdims=True)
    mt = jnp.maximum(jnp.max(t_ref[...], axis=1, keepdims=True), 0.0)
    m = _prelu(sj + msi + mt, a)
    m_ref[...] = m
    exs_ref[...] = jnp.exp(_prelu(si + sj, a) - m)


def _tc_prep(si, sj, t, a):
    return pl.pallas_call(
        _k_prep,
        grid=(1,),
        in_specs=[
            pl.BlockSpec((HEADS, N), lambda i: (0, 0)),
            pl.BlockSpec((HEADS, N), lambda i: (0, 0)),
            pl.BlockSpec((HEADS, E), lambda i: (0, 0)),
            pl.BlockSpec((1, 1), lambda i: (0, 0)),
        ],
        out_specs=[
            pl.BlockSpec((HEADS, N), lambda i: (0, 0)),
            pl.BlockSpec((HEADS, N), lambda i: (0, 0)),
        ],
        out_shape=[
            jax.ShapeDtypeStruct((HEADS, N), f32),
            jax.ShapeDtypeStruct((HEADS, N), f32),
        ],
    )(si, sj, t, a)


def _k_finish(parts_ref, exs_ref, xh_ref, inv_ref, init_ref):
    den = jnp.sum(parts_ref[...], axis=0) + exs_ref[...]
    inv = 1.0 / (den + 1e-16)
    inv_ref[...] = inv
    init_ref[...] = (exs_ref[...] * inv)[:, :, None] * xh_ref[...]


def _tc_finish(parts, exs, xh):
    return pl.pallas_call(
        _k_finish,
        grid=(1,),
        in_specs=[
            pl.BlockSpec((NW, HEADS, N), lambda i: (0, 0, 0)),
            pl.BlockSpec((HEADS, N), lambda i: (0, 0)),
            pl.BlockSpec((HEADS, N, EMB), lambda i: (0, 0, 0)),
        ],
        out_specs=[
            pl.BlockSpec((HEADS, N), lambda i: (0, 0)),
            pl.BlockSpec((HEADS, N, EMB), lambda i: (0, 0, 0)),
        ],
        out_shape=[
            jax.ShapeDtypeStruct((HEADS, N), f32),
            jax.ShapeDtypeStruct((HEADS, N, EMB), f32),
        ],
    )(parts, exs, xh)


def _k_out(aggr_ref, b_ref, lw_ref, lb_ref, ag_ref, o_ref, *, last):
    out = jnp.mean(aggr_ref[...], axis=0) + b_ref[...]
    mu = jnp.mean(out, axis=-1, keepdims=True)
    var = jnp.mean((out - mu) ** 2, axis=-1, keepdims=True)
    out = (out - mu) / jnp.sqrt(var + 1e-5) * lw_ref[...] + lb_ref[...]
    if not last:
        out = _prelu(out, ag_ref[0, 0])
    o_ref[...] = out


def _tc_out(aggr, b, lw, lb, ag, last, tn=1000):
    return pl.pallas_call(
        functools.partial(_k_out, last=last),
        grid=(N // tn,),
        in_specs=[
            pl.BlockSpec((HEADS, tn, EMB), lambda i: (0, i, 0)),
            pl.BlockSpec((1, EMB), lambda i: (0, 0)),
            pl.BlockSpec((1, EMB), lambda i: (0, 0)),
            pl.BlockSpec((1, EMB), lambda i: (0, 0)),
            pl.BlockSpec((1, 1), lambda i: (0, 0)),
        ],
        out_specs=pl.BlockSpec((tn, EMB), lambda i: (i, 0)),
        out_shape=jax.ShapeDtypeStruct((N, EMB), f32),
    )(aggr, b, lw, lb, ag)


def _sc_denom(src, dst, t, si, sj, m, a16):
    mesh = plsc.VectorSubcoreMesh(core_axis_name="c", subcore_axis_name="s")

    def body(src_h, dst_h, t_h, si_h, sj_h, m_h, a_h,
             ex_h, parts_h,
             src_v, dst_v, t_v, si_v, sj_v, m_v, den_v, ex_v, a_v):
        c = lax.axis_index("c")
        s = lax.axis_index("s")
        w = c * 16 + s
        e0 = w * EW
        pltpu.sync_copy(src_h.at[pl.ds(e0, EW)], src_v)
        pltpu.sync_copy(dst_h.at[pl.ds(e0, EW)], dst_v)
        pltpu.sync_copy(a_h, a_v)
        av = a_v[...]

        for hh in range(HEADS):
            pltpu.sync_copy(si_h.at[pl.ds(hh * N, N)], si_v)
            pltpu.sync_copy(sj_h.at[pl.ds(hh * N, N)], sj_v)
            pltpu.sync_copy(m_h.at[pl.ds(hh * N, N)], m_v)
            pltpu.sync_copy(t_h.at[pl.ds(hh * E + e0, EW)], t_v)

            def zero_body(i, _):
                den_v[pl.ds(i * 16, 16)] = jnp.zeros((16,), f32)
                return 0
            lax.fori_loop(0, N // 16, zero_body, 0)

            def edge_body(g, _):
                b = g * 16
                s16 = src_v[pl.ds(b, 16)]
                d16 = dst_v[pl.ds(b, 16)]
                tt = t_v[pl.ds(b, 16)]
                sid = plsc.load_gather(si_v, [d16])
                sjs = plsc.load_gather(sj_v, [s16])
                ms = plsc.load_gather(m_v, [s16])
                logit = sid + sjs + tt
                alpha = jnp.where(logit >= 0, logit, av * logit)
                e16 = jnp.exp(alpha - ms)
                ex_v[pl.ds(b, 16)] = e16
                plsc.addupdate_scatter(den_v, [s16], e16)
                return 0
            lax.fori_loop(0, EW // 16, edge_body, 0)

            pltpu.sync_copy(ex_v, ex_h.at[pl.ds(hh * E + e0, EW)])
            pltpu.sync_copy(den_v, parts_h.at[pl.ds((w * HEADS + hh) * N, N)])

    kern = pl.kernel(
        body,
        out_type=[
            jax.ShapeDtypeStruct((HEADS * E,), f32),
            jax.ShapeDtypeStruct((NW * HEADS * N,), f32),
        ],
        mesh=mesh,
        compiler_params=pltpu.CompilerParams(needs_layout_passes=False, use_tc_tiling_on_sc=False),
        scratch_types=[
            pltpu.VMEM((EW,), i32),
            pltpu.VMEM((EW,), i32),
            pltpu.VMEM((EW,), f32),
            pltpu.VMEM((N,), f32),
            pltpu.VMEM((N,), f32),
            pltpu.VMEM((N,), f32),
            pltpu.VMEM((N,), f32),
            pltpu.VMEM((EW,), f32),
            pltpu.VMEM((16,), f32),
        ],
    )
    return kern(src, dst, t, si, sj, m, a16)


def _sc_aggr(src, dst, ex, inv, xh2, em2, init2):
    mesh = plsc.VectorSubcoreMesh(core_axis_name="c", subcore_axis_name="s")

    def body(src_h, dst_h, ex_h, inv_h, xh_h, em_h, init_h,
             out_h,
             inv_v, wbuf,
             srcb0, srcb1, srcb2, srcb3, srcb4,
             dstb0, dstb1, dstb2, dstb3, dstb4,
             idxb0, idxb1, idxb2, idxb3, idxb4,
             exb0, exb1, exb2, exb3, exb4,
             emb0, emb1, emb2, emb3, emb4,
             outb0, outb1, outb2, outb3, outb4, spm,
             sin0, sin1, sin2, sin3, sin4,
             sbig0, sbig1, sbig2, sbig3, sbig4,
             ssc0, ssc1, ssc2, ssc3, ssc4):
        c = lax.axis_index("c")
        s = lax.axis_index("s")
        srcb = [srcb0, srcb1, srcb2, srcb3, srcb4]
        dstb = [dstb0, dstb1, dstb2, dstb3, dstb4]
        idxb = [idxb0, idxb1, idxb2, idxb3, idxb4]
        exb = [exb0, exb1, exb2, exb3, exb4]
        emb = [emb0, emb1, emb2, emb3, emb4]
        outb = [outb0, outb1, outb2, outb3, outb4]
        sin = [sin0, sin1, sin2, sin3, sin4]
        sbig = [sbig0, sbig1, sbig2, sbig3, sbig4]
        ssc = [ssc0, ssc1, ssc2, ssc3, ssc4]

        ebase = s * ET

        for ph in range(2):
            hsel = c * 2 + ph
            hN = hsel * N
            hE = hsel * E
            pltpu.sync_copy(inv_h.at[pl.ds(hN, N)], inv_v)
            pltpu.sync_copy(init_h.at[pl.ds(hN + s * NSL, NSL)],
                            spm.at[pl.ds(s * NSL, NSL)])
            plsc.subcore_barrier()

            def e_of(j):
                return ebase + j * K

            def issue_in(j, sl):
                pltpu.async_copy(src_h.at[pl.ds(e_of(j), K)], srcb[sl],
                                 sin[sl])
                pltpu.async_copy(dst_h.at[pl.ds(e_of(j), K)], dstb[sl],
                                 sin[sl])
                pltpu.async_copy(ex_h.at[pl.ds(hE + e_of(j), K)], exb[sl],
                                 sin[sl])
                pltpu.async_copy(em_h.at[pl.ds(hE + e_of(j), K)], emb[sl],
                                 sin[sl])

            def wait_in(j, sl):
                pltpu.make_async_copy(src_h.at[pl.ds(e_of(j), K)], srcb[sl],
                                      sin[sl]).wait()
                pltpu.make_async_copy(dst_h.at[pl.ds(e_of(j), K)], dstb[sl],
                                      sin[sl]).wait()
                pltpu.make_async_copy(ex_h.at[pl.ds(hE + e_of(j), K)],
                                      exb[sl], sin[sl]).wait()
                pltpu.make_async_copy(em_h.at[pl.ds(hE + e_of(j), K)],
                                      emb[sl], sin[sl]).wait()

            def build_issue_big(j, sl):
                hNv = jnp.broadcast_to(hN.astype(i32), (16,))
                for q in range(K // 16):
                    idxb[sl][pl.ds(q * 16, 16)] = (
                        srcb[sl][pl.ds(q * 16, 16)] + hNv)
                pltpu.async_copy(xh_h.at[idxb[sl]], emb[sl], sbig[sl],
                                 add=True)

            def wait_big(j, sl):
                pltpu.make_async_copy(xh_h.at[idxb[sl]], emb[sl],
                                      sbig[sl]).wait()

            def wait_sc(sl):
                pltpu.make_async_copy(outb[sl], spm.at[dstb[sl]],
                                      ssc[sl]).wait()

            def compute_scatter(j, sl):
                for q in range(K // 16):
                    b = q * 16
                    s16 = srcb[sl][pl.ds(b, 16)]
                    ex16 = exb[sl][pl.ds(b, 16)]
                    iv16 = plsc.load_gather(inv_v, [s16])
                    wbuf[pl.ds(b, 16)] = ex16 * iv16

                def row_body(r, _):
                    for u in range(2):
                        jj = r * 2 + u
                        wv = plsc.load_gather(
                            wbuf, [jnp.broadcast_to(jj, (16,))])
                        for g in range(EMB // 32):
                            pair = emb[sl][jj, pl.ds(g * 32, 32)]
                            av, bv = plsc.unpack(
                                pair, format=plsc.PackFormat.INTERLEAVED)
                            outb[sl][jj, pl.ds(g * 32, 16)] = av * wv
                            outb[sl][jj, pl.ds(g * 32 + 16, 16)] = bv * wv
                    return 0
                lax.fori_loop(0, K // 2, row_body, 0)
                pltpu.async_copy(outb[sl], spm.at[dstb[sl]], ssc[sl],
                                 add=True)

            issue_in(0, 0)
            issue_in(1, 1)
            wait_in(0, 0)
            build_issue_big(0, 0)

            def outer(o, _):
                for b5 in range(NBUF):
                    j = o * NBUF + b5
                    sl = b5
                    sl2 = (b5 + 2) % NBUF
                    sl1 = (b5 + 1) % NBUF

                    @pl.when(jnp.logical_and(j >= 3, j + 2 < NCH))
                    def _():
                        wait_sc(sl2)

                    @pl.when(j + 2 < NCH)
                    def _():
                        issue_in(j + 2, sl2)

                    @pl.when(j + 1 < NCH)
                    def _():
                        wait_in(j + 1, sl1)
                        build_issue_big(j + 1, sl1)

                    @pl.when(j < NCH)
                    def _():
                        wait_big(j, sl)
                        compute_scatter(j, sl)
                return 0
            lax.fori_loop(0, (NCH + NBUF - 1) // NBUF, outer, 0)

            for sl in range(NBUF):
                wait_sc(sl)

            plsc.subcore_barrier()
            pltpu.sync_copy(spm.at[pl.ds(s * NSL, NSL)],
                            out_h.at[pl.ds(hN + s * NSL, NSL)])
            plsc.subcore_barrier()

    kern = pl.kernel(
        body,
        out_type=[jax.ShapeDtypeStruct((HEADS * N, EMB), f32)],
        mesh=mesh,
        compiler_params=pltpu.CompilerParams(needs_layout_passes=False,
                                             use_tc_tiling_on_sc=False),
        scratch_types=(
            [pltpu.VMEM((N,), f32), pltpu.VMEM((K,), f32)]
            + [pltpu.VMEM((K,), i32) for _ in range(NBUF)]
            + [pltpu.VMEM((K,), i32) for _ in range(NBUF)]
            + [pltpu.VMEM((K,), i32) for _ in range(NBUF)]
            + [pltpu.VMEM((K,), f32) for _ in range(NBUF)]
            + [pltpu.VMEM((K, EMB), jnp.bfloat16)
               for _ in range(NBUF)]
            + [pltpu.VMEM((K, EMB), f32) for _ in range(NBUF)]
            + [pltpu.VMEM_SHARED((N, EMB), f32)]
            + [pltpu.SemaphoreType.DMA for _ in range(3 * NBUF)]
        ),
    )
    (out,) = kern(src, dst, ex, inv, xh2, em2, init2)
    return out


def kernel(x, edge_index, edge_attr, params):
    src = edge_index[0]
    dst = edge_index[1]
    ea = edge_attr.astype(f32)
    ag = params['a_gnn']

    h = _tc_embed(x, params['x_emb_W'].T, _scal(ag))

    nlayers = len(params['layers'])
    for li, p in enumerate(params['layers']):
        a = p['a']
        att = p['att'][0]
        atti = att[:, :EMB]
        attj = att[:, EMB:]
        em3, t = _tc_edge(ea, p['We_w'].T, p['We_b'][None, :], attj,
                          _scal(a))
        xh3, xhp = _tc_xh(h, p['Wlin_w'].T, p['Wlin_b'][None, :], _scal(a))
        si, sj = _tc_sij(xh3, atti, attj)
        m, exs = _tc_prep(si, sj, t, _scal(a))
        ex, parts = _sc_denom(src, dst, t.reshape(-1), si.reshape(-1),
                              sj.reshape(-1), m.reshape(-1),
                              jnp.full((16,), a, f32))
        inv, init = _tc_finish(parts.reshape(NW, HEADS, N), exs, xh3)
        aggr = _sc_aggr(src, dst, ex, inv.reshape(-1),
                        xhp.reshape(HEADS * N, EMB),
                        em3.reshape(HEADS * E, EMB),
                        init.reshape(HEADS * N, EMB))
        h = _tc_out(aggr.reshape(HEADS, N, EMB), p['bias'][None, :],
                    p['ln_w'][None, :], p['ln_b'][None, :], _scal(ag),
                    last=(li == nlayers - 1))
    return h


# speedup vs baseline: 5.1881x; 1.7347x over previous
import functools

import jax
import jax.numpy as jnp
from jax import lax
from jax.experimental import pallas as pl
from jax.experimental.pallas import tpu as pltpu
from jax.experimental.pallas import tpu_sc as plsc

N = 10000
E = 320000
EMB = 128
HEADS = 4
HD = HEADS * EMB

NW = 32
EW = E // NW
ET = E // 16
NSL = N // 16
K = 32
NCH = ET // K
NBUF = 5

f32 = jnp.float32
i32 = jnp.int32


def _prelu(z, a):
    return jnp.where(z >= 0, z, a * z)


def _scal(v):
    return jnp.reshape(v.astype(f32), (1, 1))


def _k_embed(x_ref, w_ref, a_ref, o_ref):
    o_ref[...] = _prelu(
        jnp.dot(x_ref[...], w_ref[...], preferred_element_type=f32),
        a_ref[0, 0])


def _tc_embed(x, wt, a):
    return pl.pallas_call(
        _k_embed,
        grid=(10,),
        in_specs=[
            pl.BlockSpec((N // 10, 128), lambda i: (i, 0)),
            pl.BlockSpec((128, 128), lambda i: (0, 0)),
            pl.BlockSpec((1, 1), lambda i: (0, 0)),
        ],
        out_specs=pl.BlockSpec((N // 10, 128), lambda i: (i, 0)),
        out_shape=jax.ShapeDtypeStruct((N, 128), f32),
    )(x, wt, a)


def _k_edge(ea_ref, w_ref, b_ref, attj_ref, a_ref, em_ref, t_ref):
    emt = _prelu(
        jnp.dot(ea_ref[...], w_ref[...], preferred_element_type=f32)
        + b_ref[...], a_ref[0, 0])
    em4 = emt.reshape(emt.shape[0], HEADS, EMB)
    t_ref[...] = jnp.sum(em4 * attj_ref[...][None], axis=-1).T
    em_ref[...] = jnp.transpose(em4, (1, 0, 2))


def _tc_edge(ea, wt, b, attj, a, te=2560):
    return pl.pallas_call(
        _k_edge,
        grid=(E // te,),
        in_specs=[
            pl.BlockSpec((te, 16), lambda i: (i, 0)),
            pl.BlockSpec((16, HD), lambda i: (0, 0)),
            pl.BlockSpec((1, HD), lambda i: (0, 0)),
            pl.BlockSpec((HEADS, EMB), lambda i: (0, 0)),
            pl.BlockSpec((1, 1), lambda i: (0, 0)),
        ],
        out_specs=[
            pl.BlockSpec((HEADS, te, EMB), lambda i: (0, i, 0)),
            pl.BlockSpec((HEADS, te), lambda i: (0, i)),
        ],
        out_shape=[
            jax.ShapeDtypeStruct((HEADS, E, EMB), f32),
            jax.ShapeDtypeStruct((HEADS, E), f32),
        ],
    )(ea, wt, b, attj, a)


def _k_xh(h_ref, w_ref, b_ref, a_ref, xh_ref):
    xht = _prelu(
        jnp.dot(h_ref[...], w_ref[...], preferred_element_type=f32)
        + b_ref[...], a_ref[0, 0])
    xh4 = xht.reshape(xht.shape[0], HEADS, EMB)
    xh_ref[...] = jnp.transpose(xh4, (1, 0, 2))


def _tc_xh(h, wt, b, a, tn=1000):
    return pl.pallas_call(
        _k_xh,
        grid=(N // tn,),
        in_specs=[
            pl.BlockSpec((tn, 128), lambda i: (i, 0)),
            pl.BlockSpec((128, HD), lambda i: (0, 0)),
            pl.BlockSpec((1, HD), lambda i: (0, 0)),
            pl.BlockSpec((1, 1), lambda i: (0, 0)),
        ],
        out_specs=pl.BlockSpec((HEADS, tn, EMB), lambda i: (0, i, 0)),
        out_shape=jax.ShapeDtypeStruct((HEADS, N, EMB), f32),
    )(h, wt, b, a)


def _k_sij(xh_ref, atti_ref, attj_ref, si_ref, sj_ref):
    xh = xh_ref[...]
    si_ref[...] = jnp.sum(xh * atti_ref[...][:, None, :], axis=-1)
    sj_ref[...] = jnp.sum(xh * attj_ref[...][:, None, :], axis=-1)


def _tc_sij(xh, atti, attj):
    return pl.pallas_call(
        _k_sij,
        grid=(1,),
        compiler_params=pltpu.CompilerParams(
            vmem_limit_bytes=100 * 1024 * 1024),
        in_specs=[
            pl.BlockSpec((HEADS, N, EMB), lambda i: (0, 0, 0)),
            pl.BlockSpec((HEADS, EMB), lambda i: (0, 0)),
            pl.BlockSpec((HEADS, EMB), lambda i: (0, 0)),
        ],
        out_specs=[
            pl.BlockSpec((HEADS, N), lambda i: (0, 0)),
            pl.BlockSpec((HEADS, N), lambda i: (0, 0)),
        ],
        out_shape=[
            jax.ShapeDtypeStruct((HEADS, N), f32),
            jax.ShapeDtypeStruct((HEADS, N), f32),
        ],
    )(xh, atti, attj)


def _k_prep(si_ref, sj_ref, t_ref, a_ref, m_ref, exs_ref):
    a = a_ref[0, 0]
    si = si_ref[...]
    sj = sj_ref[...]
    msi = jnp.max(si, axis=1, keepdims=True)
    mt = jnp.maximum(jnp.max(t_ref[...], axis=1, keepdims=True), 0.0)
    m = _prelu(sj + msi + mt, a)
    m_ref[...] = m
    exs_ref[...] = jnp.exp(_prelu(si + sj, a) - m)


def _tc_prep(si, sj, t, a):
    return pl.pallas_call(
        _k_prep,
        grid=(1,),
        in_specs=[
            pl.BlockSpec((HEADS, N), lambda i: (0, 0)),
            pl.BlockSpec((HEADS, N), lambda i: (0, 0)),
            pl.BlockSpec((HEADS, E), lambda i: (0, 0)),
            pl.BlockSpec((1, 1), lambda i: (0, 0)),
        ],
        out_specs=[
            pl.BlockSpec((HEADS, N), lambda i: (0, 0)),
            pl.BlockSpec((HEADS, N), lambda i: (0, 0)),
        ],
        out_shape=[
            jax.ShapeDtypeStruct((HEADS, N), f32),
            jax.ShapeDtypeStruct((HEADS, N), f32),
        ],
    )(si, sj, t, a)


def _k_finish(parts_ref, exs_ref, xh_ref, inv_ref, init_ref):
    den = jnp.sum(parts_ref[...], axis=0) + exs_ref[...]
    inv = 1.0 / (den + 1e-16)
    inv_ref[...] = inv
    init_ref[...] = (exs_ref[...] * inv)[:, :, None] * xh_ref[...]


def _tc_finish(parts, exs, xh):
    return pl.pallas_call(
        _k_finish,
        grid=(1,),
        in_specs=[
            pl.BlockSpec((NW, HEADS, N), lambda i: (0, 0, 0)),
            pl.BlockSpec((HEADS, N), lambda i: (0, 0)),
            pl.BlockSpec((HEADS, N, EMB), lambda i: (0, 0, 0)),
        ],
        out_specs=[
            pl.BlockSpec((HEADS, N), lambda i: (0, 0)),
            pl.BlockSpec((HEADS, N, EMB), lambda i: (0, 0, 0)),
        ],
        out_shape=[
            jax.ShapeDtypeStruct((HEADS, N), f32),
            jax.ShapeDtypeStruct((HEADS, N, EMB), f32),
        ],
    )(parts, exs, xh)


def _k_out(aggr_ref, b_ref, lw_ref, lb_ref, ag_ref, o_ref, *, last):
    out = jnp.mean(aggr_ref[...], axis=0) + b_ref[...]
    mu = jnp.mean(out, axis=-1, keepdims=True)
    var = jnp.mean((out - mu) ** 2, axis=-1, keepdims=True)
    out = (out - mu) / jnp.sqrt(var + 1e-5) * lw_ref[...] + lb_ref[...]
    if not last:
        out = _prelu(out, ag_ref[0, 0])
    o_ref[...] = out


def _tc_out(aggr, b, lw, lb, ag, last, tn=1000):
    return pl.pallas_call(
        functools.partial(_k_out, last=last),
        grid=(N // tn,),
        in_specs=[
            pl.BlockSpec((HEADS, tn, EMB), lambda i: (0, i, 0)),
            pl.BlockSpec((1, EMB), lambda i: (0, 0)),
            pl.BlockSpec((1, EMB), lambda i: (0, 0)),
            pl.BlockSpec((1, EMB), lambda i: (0, 0)),
            pl.BlockSpec((1, 1), lambda i: (0, 0)),
        ],
        out_specs=pl.BlockSpec((tn, EMB), lambda i: (i, 0)),
        out_shape=jax.ShapeDtypeStruct((N, EMB), f32),
    )(aggr, b, lw, lb, ag)


def _sc_denom(src, dst, t, si, sj, m, a16):
    mesh = plsc.VectorSubcoreMesh(core_axis_name="c", subcore_axis_name="s")

    def body(src_h, dst_h, t_h, si_h, sj_h, m_h, a_h,
             ex_h, parts_h,
             src_v, dst_v, t_v, si_v, sj_v, m_v, den_v, ex_v, a_v):
        c = lax.axis_index("c")
        s = lax.axis_index("s")
        w = c * 16 + s
        e0 = w * EW
        pltpu.sync_copy(src_h.at[pl.ds(e0, EW)], src_v)
        pltpu.sync_copy(dst_h.at[pl.ds(e0, EW)], dst_v)
        pltpu.sync_copy(a_h, a_v)
        av = a_v[...]

        for hh in range(HEADS):
            pltpu.sync_copy(si_h.at[pl.ds(hh * N, N)], si_v)
            pltpu.sync_copy(sj_h.at[pl.ds(hh * N, N)], sj_v)
            pltpu.sync_copy(m_h.at[pl.ds(hh * N, N)], m_v)
            pltpu.sync_copy(t_h.at[pl.ds(hh * E + e0, EW)], t_v)

            def zero_body(i, _):
                den_v[pl.ds(i * 16, 16)] = jnp.zeros((16,), f32)
                return 0
            lax.fori_loop(0, N // 16, zero_body, 0)

            def edge_body(g, _):
                b = g * 16
                s16 = src_v[pl.ds(b, 16)]
                d16 = dst_v[pl.ds(b, 16)]
                tt = t_v[pl.ds(b, 16)]
                sid = plsc.load_gather(si_v, [d16])
                sjs = plsc.load_gather(sj_v, [s16])
                ms = plsc.load_gather(m_v, [s16])
                logit = sid + sjs + tt
                alpha = jnp.where(logit >= 0, logit, av * logit)
                e16 = jnp.exp(alpha - ms)
                ex_v[pl.ds(b, 16)] = e16
                plsc.addupdate_scatter(den_v, [s16], e16)
                return 0
            lax.fori_loop(0, EW // 16, edge_body, 0)

            pltpu.sync_copy(ex_v, ex_h.at[pl.ds(hh * E + e0, EW)])
            pltpu.sync_copy(den_v, parts_h.at[pl.ds((w * HEADS + hh) * N, N)])

    kern = pl.kernel(
        body,
        out_type=[
            jax.ShapeDtypeStruct((HEADS * E,), f32),
            jax.ShapeDtypeStruct((NW * HEADS * N,), f32),
        ],
        mesh=mesh,
        compiler_params=pltpu.CompilerParams(needs_layout_passes=False, use_tc_tiling_on_sc=False),
        scratch_types=[
            pltpu.VMEM((EW,), i32),
            pltpu.VMEM((EW,), i32),
            pltpu.VMEM((EW,), f32),
            pltpu.VMEM((N,), f32),
            pltpu.VMEM((N,), f32),
            pltpu.VMEM((N,), f32),
            pltpu.VMEM((N,), f32),
            pltpu.VMEM((EW,), f32),
            pltpu.VMEM((16,), f32),
        ],
    )
    return kern(src, dst, t, si, sj, m, a16)


def _sc_aggr(src, dst, ex, inv, xh2, em2, init2):
    mesh = plsc.VectorSubcoreMesh(core_axis_name="c", subcore_axis_name="s")

    def body(src_h, dst_h, ex_h, inv_h, xh_h, em_h, init_h,
             out_h,
             inv_v, wbuf,
             srcb0, srcb1, srcb2, srcb3, srcb4,
             dstb0, dstb1, dstb2, dstb3, dstb4,
             idxb0, idxb1, idxb2, idxb3, idxb4,
             exb0, exb1, exb2, exb3, exb4,
             emb0, emb1, emb2, emb3, emb4, spm,
             sin0, sin1, sin2, sin3, sin4,
             sbig0, sbig1, sbig2, sbig3, sbig4,
             ssc0, ssc1, ssc2, ssc3, ssc4):
        c = lax.axis_index("c")
        s = lax.axis_index("s")
        srcb = [srcb0, srcb1, srcb2, srcb3, srcb4]
        dstb = [dstb0, dstb1, dstb2, dstb3, dstb4]
        idxb = [idxb0, idxb1, idxb2, idxb3, idxb4]
        exb = [exb0, exb1, exb2, exb3, exb4]
        emb = [emb0, emb1, emb2, emb3, emb4]
        sin = [sin0, sin1, sin2, sin3, sin4]
        sbig = [sbig0, sbig1, sbig2, sbig3, sbig4]
        ssc = [ssc0, ssc1, ssc2, ssc3, ssc4]

        ebase = s * ET

        for ph in range(2):
            hsel = c * 2 + ph
            hN = hsel * N
            hE = hsel * E
            pltpu.sync_copy(inv_h.at[pl.ds(hN, N)], inv_v)
            pltpu.sync_copy(init_h.at[pl.ds(hN + s * NSL, NSL)],
                            spm.at[pl.ds(s * NSL, NSL)])
            plsc.subcore_barrier()

            def e_of(j):
                return ebase + j * K

            def issue_in(j, sl):
                pltpu.async_copy(src_h.at[pl.ds(e_of(j), K)], srcb[sl],
                                 sin[sl])
                pltpu.async_copy(dst_h.at[pl.ds(e_of(j), K)], dstb[sl],
                                 sin[sl])
                pltpu.async_copy(ex_h.at[pl.ds(hE + e_of(j), K)], exb[sl],
                                 sin[sl])
                pltpu.async_copy(em_h.at[pl.ds(hE + e_of(j), K)], emb[sl],
                                 sin[sl])

            def wait_in(j, sl):
                pltpu.make_async_copy(src_h.at[pl.ds(e_of(j), K)], srcb[sl],
                                      sin[sl]).wait()
                pltpu.make_async_copy(dst_h.at[pl.ds(e_of(j), K)], dstb[sl],
                                      sin[sl]).wait()
                pltpu.make_async_copy(ex_h.at[pl.ds(hE + e_of(j), K)],
                                      exb[sl], sin[sl]).wait()
                pltpu.make_async_copy(em_h.at[pl.ds(hE + e_of(j), K)],
                                      emb[sl], sin[sl]).wait()

            def build_issue_big(j, sl):
                hNv = jnp.broadcast_to(hN.astype(i32), (16,))
                for q in range(K // 16):
                    idxb[sl][pl.ds(q * 16, 16)] = (
                        srcb[sl][pl.ds(q * 16, 16)] + hNv)
                pltpu.async_copy(xh_h.at[idxb[sl]], emb[sl], sbig[sl],
                                 add=True)

            def wait_big(j, sl):
                pltpu.make_async_copy(xh_h.at[idxb[sl]], emb[sl],
                                      sbig[sl]).wait()

            def wait_sc(sl):
                pltpu.make_async_copy(emb[sl], spm.at[dstb[sl]],
                                      ssc[sl]).wait()

            def compute_scatter(j, sl):
                for q in range(K // 16):
                    b = q * 16
                    s16 = srcb[sl][pl.ds(b, 16)]
                    ex16 = exb[sl][pl.ds(b, 16)]
                    iv16 = plsc.load_gather(inv_v, [s16])
                    wbuf[pl.ds(b, 16)] = ex16 * iv16

                def row_body(r, _):
                    for u in range(4):
                        jj = r * 4 + u
                        wv = plsc.load_gather(
                            wbuf, [jnp.broadcast_to(jj, (16,))])
                        for q in range(EMB // 16):
                            sl_q = pl.ds(q * 16, 16)
                            emb[sl][jj, sl_q] = emb[sl][jj, sl_q] * wv
                    return 0
                lax.fori_loop(0, K // 4, row_body, 0)
                pltpu.async_copy(emb[sl], spm.at[dstb[sl]], ssc[sl],
                                 add=True)

            issue_in(0, 0)
            issue_in(1, 1)
            wait_in(0, 0)
            build_issue_big(0, 0)

            def outer(o, _):
                for b5 in range(NBUF):
                    j = o * NBUF + b5
                    sl = b5
                    sl2 = (b5 + 2) % NBUF
                    sl1 = (b5 + 1) % NBUF

                    @pl.when(jnp.logical_and(j >= 3, j + 2 < NCH))
                    def _():
                        wait_sc(sl2)

                    @pl.when(j + 2 < NCH)
                    def _():
                        issue_in(j + 2, sl2)

                    @pl.when(j + 1 < NCH)
                    def _():
                        wait_in(j + 1, sl1)
                        build_issue_big(j + 1, sl1)

                    @pl.when(j < NCH)
                    def _():
                        wait_big(j, sl)
                        compute_scatter(j, sl)
                return 0
            lax.fori_loop(0, (NCH + NBUF - 1) // NBUF, outer, 0)

            for sl in range(NBUF):
                wait_sc(sl)

            plsc.subcore_barrier()
            pltpu.sync_copy(spm.at[pl.ds(s * NSL, NSL)],
                            out_h.at[pl.ds(hN + s * NSL, NSL)])
            plsc.subcore_barrier()

    kern = pl.kernel(
        body,
        out_type=[jax.ShapeDtypeStruct((HEADS * N, EMB), f32)],
        mesh=mesh,
        compiler_params=pltpu.CompilerParams(needs_layout_passes=False,
                                             use_tc_tiling_on_sc=False),
        scratch_types=(
            [pltpu.VMEM((N,), f32), pltpu.VMEM((K,), f32)]
            + [pltpu.VMEM((K,), i32) for _ in range(NBUF)]
            + [pltpu.VMEM((K,), i32) for _ in range(NBUF)]
            + [pltpu.VMEM((K,), i32) for _ in range(NBUF)]
            + [pltpu.VMEM((K,), f32) for _ in range(NBUF)]
            + [pltpu.VMEM((K, EMB), f32) for _ in range(NBUF)]
            + [pltpu.VMEM_SHARED((N, EMB), f32)]
            + [pltpu.SemaphoreType.DMA for _ in range(3 * NBUF)]
        ),
    )
    (out,) = kern(src, dst, ex, inv, xh2, em2, init2)
    return out


def kernel(x, edge_index, edge_attr, params):
    src = edge_index[0]
    dst = edge_index[1]
    ea = edge_attr.astype(f32)
    ag = params['a_gnn']

    h = _tc_embed(x, params['x_emb_W'].T, _scal(ag))

    nlayers = len(params['layers'])
    for li, p in enumerate(params['layers']):
        a = p['a']
        att = p['att'][0]
        atti = att[:, :EMB]
        attj = att[:, EMB:]
        em3, t = _tc_edge(ea, p['We_w'].T, p['We_b'][None, :], attj,
                          _scal(a))
        xh3 = _tc_xh(h, p['Wlin_w'].T, p['Wlin_b'][None, :], _scal(a))
        si, sj = _tc_sij(xh3, atti, attj)
        m, exs = _tc_prep(si, sj, t, _scal(a))
        ex, parts = _sc_denom(src, dst, t.reshape(-1), si.reshape(-1),
                              sj.reshape(-1), m.reshape(-1),
                              jnp.full((16,), a, f32))
        inv, init = _tc_finish(parts.reshape(NW, HEADS, N), exs, xh3)
        aggr = _sc_aggr(src, dst, ex, inv.reshape(-1),
                        xh3.reshape(HEADS * N, EMB),
                        em3.reshape(HEADS * E, EMB),
                        init.reshape(HEADS * N, EMB))
        h = _tc_out(aggr.reshape(HEADS, N, EMB), p['bias'][None, :],
                    p['ln_w'][None, :], p['ln_b'][None, :], _scal(ag),
                    last=(li == nlayers - 1))
    return h
